# baseline (speedup 1.0000x reference)
import functools

import jax
import jax.numpy as jnp
from jax import lax
from jax.experimental import pallas as pl
from jax.experimental.pallas import tpu as pltpu
from jax.experimental.pallas import tpu_sc as plsc

N_NODES = 10000
N_EDGES = 320000
DIM = 128

NC = 2
NS = 16
NW = NC * NS
SUB = 128
CH = 2560
E_PAD = CH * SUB
S0 = 80
S1 = 160 - S0
S_MAX = max(S0, S1)
S_ONES = CH // NW
NRANGE = N_NODES // 2
SLAB = 320
N_SP = NS * SLAB
ROWB = 1000
RB_PER_RANGE = NRANGE // ROWB
GRID = N_NODES // ROWB

_MESH = plsc.VectorSubcoreMesh(core_axis_name="c", subcore_axis_name="s")


def _wid_sid():
    cid = lax.axis_index("c")
    sid = lax.axis_index("s")
    return cid, sid, cid * NS + sid


def _sc_agg_phases(steps, start, h_hbm, src_hbm, dst0_hbm, dst1_hbm, z_hbm,
                   s_out, src_v, dst_v0, dst_v1, rows_a, rows_b, agg_sh,
                   sem_a, sem_b, cid, sid):
    pltpu.sync_copy(src_hbm.at[pl.ds(start, steps)], src_v.at[pl.ds(0, steps)])
    pltpu.sync_copy(dst0_hbm.at[pl.ds(start, steps)], dst_v0.at[pl.ds(0, steps)])
    pltpu.sync_copy(dst1_hbm.at[pl.ds(start, steps)], dst_v1.at[pl.ds(0, steps)])

    for rng, dst_v in enumerate((dst_v0, dst_v1)):
        pltpu.sync_copy(z_hbm, agg_sh.at[pl.ds(sid * SLAB, SLAB)])
        plsc.subcore_barrier()

        pltpu.async_copy(h_hbm.at[src_v.at[0]], rows_a, sem_a)

        def step(j, _):
            def do(buf, sem, obuf, osem):
                @pl.when(j + 1 < steps)
                def _():
                    pltpu.async_copy(h_hbm.at[src_v.at[j + 1]], obuf, osem)
                pltpu.make_async_copy(h_hbm.at[src_v.at[j]], buf, sem).wait()
                pltpu.sync_copy(buf, agg_sh.at[dst_v.at[j]], add=True)

            cur = jnp.remainder(j, 2)

            @pl.when(cur == 0)
            def _():
                do(rows_a, sem_a, rows_b, sem_b)

            @pl.when(cur == 1)
            def _():
                do(rows_b, sem_b, rows_a, sem_a)
            return 0

        lax.fori_loop(0, steps, step, 0)
        plsc.subcore_barrier()

        obase = pl.multiple_of(jnp.minimum(sid * SLAB, NRANGE - SLAB), 8)
        pltpu.sync_copy(agg_sh.at[pl.ds(obase, SLAB)],
                        s_out.at[rng].at[cid].at[pl.ds(obase, SLAB)])
        plsc.subcore_barrier()


def _sc_agg_body(h_hbm, src_hbm, dst0_hbm, dst1_hbm, z_hbm, s_out,
                 src_v, dst_v0, dst_v1, rows_a, rows_b, agg_sh, sem_a, sem_b):
    cid, sid, wid = _wid_sid()
    args = (h_hbm, src_hbm, dst0_hbm, dst1_hbm, z_hbm, s_out,
            src_v, dst_v0, dst_v1, rows_a, rows_b, agg_sh, sem_a, sem_b,
            cid, sid)

    @pl.when(cid == 0)
    def _():
        _sc_agg_phases(S0, sid * S0, *args)

    @pl.when(cid == 1)
    def _():
        _sc_agg_phases(S1, NS * S0 + sid * S1, *args)


_sc_agg = pl.kernel(
    _sc_agg_body,
    out_type=jax.ShapeDtypeStruct((2, NC, NRANGE, DIM), jnp.float32),
    mesh=_MESH,
    scratch_types=[
        pltpu.VMEM((S_MAX, SUB), jnp.int32),
        pltpu.VMEM((S_MAX, SUB), jnp.int32),
        pltpu.VMEM((S_MAX, SUB), jnp.int32),
        pltpu.VMEM((SUB, DIM), jnp.float32),
        pltpu.VMEM((SUB, DIM), jnp.float32),
        pltpu.VMEM_SHARED((N_SP, DIM), jnp.float32),
        pltpu.SemaphoreType.DMA,
        pltpu.SemaphoreType.DMA,
    ],
)

def _sc_ones_body(ones_hbm, dst0_hbm, dst1_hbm, z_hbm, s_out,
                  dst_v0, dst_v1, rows_a, agg_sh):
    cid, sid, wid = _wid_sid()
    pltpu.sync_copy(ones_hbm, rows_a)
    pltpu.sync_copy(dst0_hbm.at[pl.ds(wid * S_ONES, S_ONES)], dst_v0)
    pltpu.sync_copy(dst1_hbm.at[pl.ds(wid * S_ONES, S_ONES)], dst_v1)

    for rng, dst_v in enumerate((dst_v0, dst_v1)):
        pltpu.sync_copy(z_hbm, agg_sh.at[pl.ds(sid * SLAB, SLAB)])
        plsc.subcore_barrier()

        def step(j, _):
            pltpu.sync_copy(rows_a, agg_sh.at[dst_v.at[j]], add=True)
            return 0

        lax.fori_loop(0, S_ONES, step, 0)
        plsc.subcore_barrier()
        obase = pl.multiple_of(jnp.minimum(sid * SLAB, NRANGE - SLAB), 8)
        pltpu.sync_copy(agg_sh.at[pl.ds(obase, SLAB)],
                        s_out.at[rng].at[cid].at[pl.ds(obase, SLAB)])
        plsc.subcore_barrier()


_sc_ones = pl.kernel(
    _sc_ones_body,
    out_type=jax.ShapeDtypeStruct((2, NC, NRANGE, DIM), jnp.float32),
    mesh=_MESH,
    scratch_types=[
        pltpu.VMEM((S_ONES, SUB), jnp.int32),
        pltpu.VMEM((S_ONES, SUB), jnp.int32),
        pltpu.VMEM((SUB, DIM), jnp.float32),
        pltpu.VMEM_SHARED((N_SP, DIM), jnp.float32),
    ],
)


def _tc_body(relu, s_ref, d_ref, w_ref, b_ref, o_ref):
    s = s_ref[0, 0] + s_ref[0, 1]
    deg = d_ref[0, 0, :, 0:1] + d_ref[0, 1, :, 0:1]
    t = s * (1.0 / jnp.maximum(deg, 1.0))
    h = jnp.dot(t, w_ref[...], preferred_element_type=jnp.float32) + b_ref[...]
    o_ref[...] = jnp.maximum(h, 0.0) if relu else h


def _make_tc_pass(relu: bool):
    return pl.pallas_call(
        functools.partial(_tc_body, relu),
        grid=(GRID,),
        in_specs=[
            pl.BlockSpec((1, NC, ROWB, DIM),
                         lambda i: (i // RB_PER_RANGE, 0, i % RB_PER_RANGE, 0)),
            pl.BlockSpec((1, NC, ROWB, DIM),
                         lambda i: (i // RB_PER_RANGE, 0, i % RB_PER_RANGE, 0)),
            pl.BlockSpec((DIM, DIM), lambda i: (0, 0)),
            pl.BlockSpec((1, DIM), lambda i: (0, 0)),
        ],
        out_specs=pl.BlockSpec((ROWB, DIM), lambda i: (i, 0)),
        out_shape=jax.ShapeDtypeStruct((N_NODES, DIM), jnp.float32),
    )


_tc_pass_relu = _make_tc_pass(relu=True)
_tc_pass_lin = _make_tc_pass(relu=False)


def kernel(x, edge_index, W1, b1, W2, b2):
    ei = edge_index.astype(jnp.int32)
    pad = E_PAD - N_EDGES
    src = jnp.concatenate([ei[0], jnp.zeros((pad,), jnp.int32)])
    dstp = jnp.concatenate([ei[1], jnp.full((pad,), N_NODES, jnp.int32)])
    dummy = NRANGE + jnp.arange(E_PAD, dtype=jnp.int32) % (N_SP - NRANGE)
    dst0 = jnp.where(dstp < NRANGE, dstp, dummy)
    t = dstp - NRANGE
    dst1 = jnp.where((t < 0) | (t >= NRANGE), dummy, t)
    src3 = src.reshape(CH, SUB)
    dst0_3 = dst0.reshape(CH, SUB)
    dst1_3 = dst1.reshape(CH, SUB)
    z128 = jnp.zeros((SLAB, DIM), jnp.float32)
    ones128 = jnp.ones((SUB, DIM), jnp.float32)

    deg = _sc_ones(ones128, dst0_3, dst1_3, z128)
    s1 = _sc_agg(x, src3, dst0_3, dst1_3, z128)
    h1 = _tc_pass_relu(s1, deg, W1, b1.reshape(1, DIM))
    s2 = _sc_agg(h1, src3, dst0_3, dst1_3, z128)
    out = _tc_pass_lin(s2, deg, W2, b2.reshape(1, DIM))
    return out


# speedup vs baseline: 1.5868x; 1.5868x over previous
import functools

import jax
import jax.numpy as jnp
from jax import lax
from jax.experimental import pallas as pl
from jax.experimental.pallas import tpu as pltpu
from jax.experimental.pallas import tpu_sc as plsc

N_NODES = 10000
N_EDGES = 320000
DIM = 128

NC = 2
NS = 16
NW = NC * NS
SUB = 128
STEPS = -(-N_EDGES // (NW * SUB))
E_PAD = NW * SUB * STEPS
NRANGE = N_NODES // 2
SLAB = 320
N_SP = NS * SLAB
ROWB = 1000
RB_PER_RANGE = NRANGE // ROWB
GRID = N_NODES // ROWB

_MESH = plsc.VectorSubcoreMesh(core_axis_name="c", subcore_axis_name="s")


def _wid_sid():
    cid = lax.axis_index("c")
    sid = lax.axis_index("s")
    return cid, sid, cid * NS + sid


def _sc_agg_body(h_hbm, src_hbm, dst0_hbm, dst1_hbm, z_hbm, s_out,
                 src_v, dst_v0, dst_v1, rows_a, rows_b, agg_sh, sem_a, sem_b):
    cid, sid, wid = _wid_sid()

    pltpu.sync_copy(src_hbm.at[wid], src_v)
    pltpu.sync_copy(dst0_hbm.at[wid], dst_v0)
    pltpu.sync_copy(dst1_hbm.at[wid], dst_v1)

    for rng, dst_v in enumerate((dst_v0, dst_v1)):
        pltpu.sync_copy(z_hbm, agg_sh.at[pl.ds(sid * SLAB, SLAB)])
        plsc.subcore_barrier()

        pltpu.async_copy(h_hbm.at[src_v.at[0]], rows_a, sem_a)

        def step(j, _):
            def do(buf, sem, obuf, osem):
                @pl.when(j + 1 < STEPS)
                def _():
                    pltpu.async_copy(h_hbm.at[src_v.at[j + 1]], obuf, osem)
                pltpu.make_async_copy(h_hbm.at[src_v.at[j]], buf, sem).wait()
                pltpu.sync_copy(buf, agg_sh.at[dst_v.at[j]], add=True)

            cur = jnp.remainder(j, 2)

            @pl.when(cur == 0)
            def _():
                do(rows_a, sem_a, rows_b, sem_b)

            @pl.when(cur == 1)
            def _():
                do(rows_b, sem_b, rows_a, sem_a)
            return 0

        lax.fori_loop(0, STEPS, step, 0)
        plsc.subcore_barrier()

        obase = pl.multiple_of(jnp.minimum(sid * SLAB, NRANGE - SLAB), 8)
        pltpu.sync_copy(agg_sh.at[pl.ds(obase, SLAB)],
                        s_out.at[rng].at[cid].at[pl.ds(obase, SLAB)])
        plsc.subcore_barrier()


_sc_agg = pl.kernel(
    _sc_agg_body,
    out_type=jax.ShapeDtypeStruct((2, NC, NRANGE, DIM), jnp.float32),
    mesh=_MESH,
    scratch_types=[
        pltpu.VMEM((STEPS, SUB), jnp.int32),
        pltpu.VMEM((STEPS, SUB), jnp.int32),
        pltpu.VMEM((STEPS, SUB), jnp.int32),
        pltpu.VMEM((SUB, DIM), jnp.float32),
        pltpu.VMEM((SUB, DIM), jnp.float32),
        pltpu.VMEM_SHARED((N_SP, DIM), jnp.float32),
        pltpu.SemaphoreType.DMA,
        pltpu.SemaphoreType.DMA,
    ],
)


def _sc_ones_body(ones_hbm, dst0_hbm, dst1_hbm, z_hbm, s_out,
                  dst_v0, dst_v1, rows_a, agg_sh):
    cid, sid, wid = _wid_sid()
    pltpu.sync_copy(ones_hbm, rows_a)
    pltpu.sync_copy(dst0_hbm.at[wid], dst_v0)
    pltpu.sync_copy(dst1_hbm.at[wid], dst_v1)

    for rng, dst_v in enumerate((dst_v0, dst_v1)):
        pltpu.sync_copy(z_hbm, agg_sh.at[pl.ds(sid * SLAB, SLAB)])
        plsc.subcore_barrier()

        def step(j, _):
            pltpu.sync_copy(rows_a, agg_sh.at[dst_v.at[j]], add=True)
            return 0

        lax.fori_loop(0, STEPS, step, 0)
        plsc.subcore_barrier()
        obase = pl.multiple_of(jnp.minimum(sid * SLAB, NRANGE - SLAB), 8)
        pltpu.sync_copy(agg_sh.at[pl.ds(obase, SLAB)],
                        s_out.at[rng].at[cid].at[pl.ds(obase, SLAB)])
        plsc.subcore_barrier()


_sc_ones = pl.kernel(
    _sc_ones_body,
    out_type=jax.ShapeDtypeStruct((2, NC, NRANGE, DIM), jnp.float32),
    mesh=_MESH,
    scratch_types=[
        pltpu.VMEM((STEPS, SUB), jnp.int32),
        pltpu.VMEM((STEPS, SUB), jnp.int32),
        pltpu.VMEM((SUB, DIM), jnp.float32),
        pltpu.VMEM_SHARED((N_SP, DIM), jnp.float32),
    ],
)


def _tc_body(relu, s_ref, d_ref, w_ref, b_ref, o_ref):
    s = s_ref[0, 0] + s_ref[0, 1]
    deg = d_ref[0, 0, :, 0:1] + d_ref[0, 1, :, 0:1]
    t = s * (1.0 / jnp.maximum(deg, 1.0))
    h = jnp.dot(t, w_ref[...], preferred_element_type=jnp.float32) + b_ref[...]
    o_ref[...] = jnp.maximum(h, 0.0) if relu else h


def _make_tc_pass(relu: bool):
    return pl.pallas_call(
        functools.partial(_tc_body, relu),
        grid=(GRID,),
        in_specs=[
            pl.BlockSpec((1, NC, ROWB, DIM),
                         lambda i: (i // RB_PER_RANGE, 0, i % RB_PER_RANGE, 0)),
            pl.BlockSpec((1, NC, ROWB, DIM),
                         lambda i: (i // RB_PER_RANGE, 0, i % RB_PER_RANGE, 0)),
            pl.BlockSpec((DIM, DIM), lambda i: (0, 0)),
            pl.BlockSpec((1, DIM), lambda i: (0, 0)),
        ],
        out_specs=pl.BlockSpec((ROWB, DIM), lambda i: (i, 0)),
        out_shape=jax.ShapeDtypeStruct((N_NODES, DIM), jnp.float32),
    )


_tc_pass_relu = _make_tc_pass(relu=True)
_tc_pass_lin = _make_tc_pass(relu=False)


def kernel(x, edge_index, W1, b1, W2, b2):
    ei = edge_index.astype(jnp.int32)
    pad = E_PAD - N_EDGES
    src = jnp.concatenate([ei[0], jnp.zeros((pad,), jnp.int32)])
    dstp = jnp.concatenate([ei[1], jnp.full((pad,), N_NODES, jnp.int32)])
    dummy = NRANGE + jnp.arange(E_PAD, dtype=jnp.int32) % (N_SP - NRANGE)
    dst0 = jnp.where(dstp < NRANGE, dstp, dummy)
    t = dstp - NRANGE
    dst1 = jnp.where((t < 0) | (t >= NRANGE), dummy, t)
    src3 = src.reshape(NW, STEPS, SUB)
    dst0_3 = dst0.reshape(NW, STEPS, SUB)
    dst1_3 = dst1.reshape(NW, STEPS, SUB)
    z128 = jnp.zeros((SLAB, DIM), jnp.float32)
    ones128 = jnp.ones((SUB, DIM), jnp.float32)

    deg = _sc_ones(ones128, dst0_3, dst1_3, z128)
    s1 = _sc_agg(x, src3, dst0_3, dst1_3, z128)
    h1 = _tc_pass_relu(s1, deg, W1, b1.reshape(1, DIM))
    s2 = _sc_agg(h1, src3, dst0_3, dst1_3, z128)
    out = _tc_pass_lin(s2, deg, W2, b2.reshape(1, DIM))
    return out


# speedup vs baseline: 1.5992x; 1.0078x over previous
import functools

import jax
import jax.numpy as jnp
from jax import lax
from jax.experimental import pallas as pl
from jax.experimental.pallas import tpu as pltpu
from jax.experimental.pallas import tpu_sc as plsc

N_NODES = 10000
N_EDGES = 320000
DIM = 128

NC = 2
NS = 16
NW = NC * NS
SUB = 128
STEPS = -(-N_EDGES // (NW * SUB))
E_PAD = NW * SUB * STEPS
NRANGE = N_NODES // 2
SLAB = 320
N_SP = NS * SLAB
ROWB = 1000
RB_PER_RANGE = NRANGE // ROWB
GRID = N_NODES // ROWB

_MESH = plsc.VectorSubcoreMesh(core_axis_name="c", subcore_axis_name="s")


def _wid_sid():
    cid = lax.axis_index("c")
    sid = lax.axis_index("s")
    return cid, sid, cid * NS + sid


def _sc_agg_body(h_hbm, src_hbm, dst0_hbm, dst1_hbm, z_hbm, s_out,
                 src_v, dst_v0, dst_v1, rows_a, rows_b, rows_c, agg_sh,
                 sem_a, sem_b, sem_c):
    cid, sid, wid = _wid_sid()

    pltpu.sync_copy(src_hbm.at[wid], src_v)
    pltpu.sync_copy(dst0_hbm.at[wid], dst_v0)
    pltpu.sync_copy(dst1_hbm.at[wid], dst_v1)

    for rng, dst_v in enumerate((dst_v0, dst_v1)):
        pltpu.sync_copy(z_hbm, agg_sh.at[pl.ds(sid * SLAB, SLAB)])
        plsc.subcore_barrier()

        pltpu.async_copy(h_hbm.at[src_v.at[0]], rows_a, sem_a)
        pltpu.async_copy(h_hbm.at[src_v.at[1]], rows_b, sem_b)

        def step(j, _):
            def do(buf, sem, pbuf, psem):
                @pl.when(j + 2 < STEPS)
                def _():
                    pltpu.async_copy(h_hbm.at[src_v.at[j + 2]], pbuf, psem)
                pltpu.make_async_copy(h_hbm.at[src_v.at[j]], buf, sem).wait()
                pltpu.sync_copy(buf, agg_sh.at[dst_v.at[j]], add=True)

            cur = jnp.remainder(j, 3)

            @pl.when(cur == 0)
            def _():
                do(rows_a, sem_a, rows_c, sem_c)

            @pl.when(cur == 1)
            def _():
                do(rows_b, sem_b, rows_a, sem_a)

            @pl.when(cur == 2)
            def _():
                do(rows_c, sem_c, rows_b, sem_b)
            return 0

        lax.fori_loop(0, STEPS, step, 0)
        plsc.subcore_barrier()

        obase = pl.multiple_of(jnp.minimum(sid * SLAB, NRANGE - SLAB), 8)
        pltpu.sync_copy(agg_sh.at[pl.ds(obase, SLAB)],
                        s_out.at[rng].at[cid].at[pl.ds(obase, SLAB)])
        plsc.subcore_barrier()


_sc_agg = pl.kernel(
    _sc_agg_body,
    out_type=jax.ShapeDtypeStruct((2, NC, NRANGE, DIM), jnp.float32),
    mesh=_MESH,
    scratch_types=[
        pltpu.VMEM((STEPS, SUB), jnp.int32),
        pltpu.VMEM((STEPS, SUB), jnp.int32),
        pltpu.VMEM((STEPS, SUB), jnp.int32),
        pltpu.VMEM((SUB, DIM), jnp.float32),
        pltpu.VMEM((SUB, DIM), jnp.float32),
        pltpu.VMEM((SUB, DIM), jnp.float32),
        pltpu.VMEM_SHARED((N_SP, DIM), jnp.float32),
        pltpu.SemaphoreType.DMA,
        pltpu.SemaphoreType.DMA,
        pltpu.SemaphoreType.DMA,
    ],
)


def _sc_ones_body(ones_hbm, dst0_hbm, dst1_hbm, z_hbm, s_out,
                  dst_v0, dst_v1, rows_a, agg_sh):
    cid, sid, wid = _wid_sid()
    pltpu.sync_copy(ones_hbm, rows_a)
    pltpu.sync_copy(dst0_hbm.at[wid], dst_v0)
    pltpu.sync_copy(dst1_hbm.at[wid], dst_v1)

    for rng, dst_v in enumerate((dst_v0, dst_v1)):
        pltpu.sync_copy(z_hbm, agg_sh.at[pl.ds(sid * SLAB, SLAB)])
        plsc.subcore_barrier()

        def step(j, _):
            pltpu.sync_copy(rows_a, agg_sh.at[dst_v.at[j]], add=True)
            return 0

        lax.fori_loop(0, STEPS, step, 0)
        plsc.subcore_barrier()
        obase = pl.multiple_of(jnp.minimum(sid * SLAB, NRANGE - SLAB), 8)
        pltpu.sync_copy(agg_sh.at[pl.ds(obase, SLAB)],
                        s_out.at[rng].at[cid].at[pl.ds(obase, SLAB)])
        plsc.subcore_barrier()


_sc_ones = pl.kernel(
    _sc_ones_body,
    out_type=jax.ShapeDtypeStruct((2, NC, NRANGE, DIM), jnp.float32),
    mesh=_MESH,
    scratch_types=[
        pltpu.VMEM((STEPS, SUB), jnp.int32),
        pltpu.VMEM((STEPS, SUB), jnp.int32),
        pltpu.VMEM((SUB, DIM), jnp.float32),
        pltpu.VMEM_SHARED((N_SP, DIM), jnp.float32),
    ],
)


def _tc_body(relu, s_ref, d_ref, w_ref, b_ref, o_ref):
    s = s_ref[0, 0] + s_ref[0, 1]
    deg = d_ref[0, 0, :, 0:1] + d_ref[0, 1, :, 0:1]
    t = s * (1.0 / jnp.maximum(deg, 1.0))
    h = jnp.dot(t, w_ref[...], preferred_element_type=jnp.float32) + b_ref[...]
    o_ref[...] = jnp.maximum(h, 0.0) if relu else h


def _make_tc_pass(relu: bool):
    return pl.pallas_call(
        functools.partial(_tc_body, relu),
        grid=(GRID,),
        in_specs=[
            pl.BlockSpec((1, NC, ROWB, DIM),
                         lambda i: (i // RB_PER_RANGE, 0, i % RB_PER_RANGE, 0)),
            pl.BlockSpec((1, NC, ROWB, DIM),
                         lambda i: (i // RB_PER_RANGE, 0, i % RB_PER_RANGE, 0)),
            pl.BlockSpec((DIM, DIM), lambda i: (0, 0)),
            pl.BlockSpec((1, DIM), lambda i: (0, 0)),
        ],
        out_specs=pl.BlockSpec((ROWB, DIM), lambda i: (i, 0)),
        out_shape=jax.ShapeDtypeStruct((N_NODES, DIM), jnp.float32),
    )


_tc_pass_relu = _make_tc_pass(relu=True)
_tc_pass_lin = _make_tc_pass(relu=False)


def kernel(x, edge_index, W1, b1, W2, b2):
    ei = edge_index.astype(jnp.int32)
    pad = E_PAD - N_EDGES
    src = jnp.concatenate([ei[0], jnp.zeros((pad,), jnp.int32)])
    dstp = jnp.concatenate([ei[1], jnp.full((pad,), N_NODES, jnp.int32)])
    dummy = NRANGE + jnp.arange(E_PAD, dtype=jnp.int32) % (N_SP - NRANGE)
    dst0 = jnp.where(dstp < NRANGE, dstp, dummy)
    t = dstp - NRANGE
    dst1 = jnp.where((t < 0) | (t >= NRANGE), dummy, t)
    src3 = src.reshape(NW, STEPS, SUB)
    dst0_3 = dst0.reshape(NW, STEPS, SUB)
    dst1_3 = dst1.reshape(NW, STEPS, SUB)
    z128 = jnp.zeros((SLAB, DIM), jnp.float32)
    ones128 = jnp.ones((SUB, DIM), jnp.float32)

    deg = _sc_ones(ones128, dst0_3, dst1_3, z128)
    s1 = _sc_agg(x, src3, dst0_3, dst1_3, z128)
    h1 = _tc_pass_relu(s1, deg, W1, b1.reshape(1, DIM))
    s2 = _sc_agg(h1, src3, dst0_3, dst1_3, z128)
    out = _tc_pass_lin(s2, deg, W2, b2.reshape(1, DIM))
    return out
